# baseline (speedup 1.0000x reference)
import functools

import jax
import jax.numpy as jnp
from jax import lax
from jax.experimental import pallas as pl
from jax.experimental.pallas import tpu as pltpu
from jax.experimental.pallas import tpu_sc as plsc

NC = 2
NS = 16
L = 16
CH = 128


def _ceil_to(x, m):
  return (x + m - 1) // m * m


def _make_seg_sum(N, D, E_pad, ch, with_counts):
  NW = NC * NS
  N_ACC = _ceil_to(N + 1, NS * L)
  R = N_ACC // NS
  T = E_pad // (NW * ch)
  assert E_pad % (NW * ch) == 0

  mesh = plsc.VectorSubcoreMesh(
      core_axis_name="c", subcore_axis_name="s",
      num_cores=NC, num_subcores=NS)

  WB = min(ch, 128)
  assert R % WB == 0
  out_type = [jax.ShapeDtypeStruct((NC * N_ACC, D), jnp.float32)]
  scratch = [
      pltpu.VMEM((ch,), jnp.int32),
      pltpu.VMEM((ch,), jnp.int32),
      pltpu.VMEM((ch, D), jnp.float32),
      pltpu.VMEM_SHARED((N_ACC, D), jnp.float32),
      pltpu.SemaphoreType.DMA,
  ]
  if with_counts:
    out_type.append(jax.ShapeDtypeStruct((NC * N_ACC, L), jnp.float32))
    scratch += [
        pltpu.VMEM((ch, L), jnp.float32),
        pltpu.VMEM_SHARED((N_ACC, L), jnp.float32),
    ]

  def body(x_hbm, src_hbm, dst_hbm, *refs):
    if with_counts:
      (psum, pcnt, idx_s, idx_d, rows, acc, sem, ones, cacc) = refs
    else:
      (psum, idx_s, idx_d, rows, acc, sem) = refs
    c = lax.axis_index("c")
    s = lax.axis_index("s")
    wk = s * NC + c
    base = s * R
    obase = c * N_ACC + base
    zero = jnp.zeros((L,), jnp.float32)
    one = jnp.ones((L,), jnp.float32)

    @pl.loop(0, WB)
    def _zfill(i):
      for j in range(D // L):
        rows[i, pl.ds(j * L, L)] = zero

    @pl.loop(0, R // WB)
    def _zero(k):
      pltpu.sync_copy(rows.at[pl.ds(0, WB)], acc.at[pl.ds(base + k * WB, WB)])

    if with_counts:
      @pl.loop(0, ch)
      def _czfill(i):
        ones[i, pl.ds(0, L)] = zero

      @pl.loop(0, R // WB)
      def _czero(k):
        pltpu.sync_copy(ones.at[pl.ds(0, WB)],
                        cacc.at[pl.ds(base + k * WB, WB)])

      @pl.loop(0, ch)
      def _ofill(i):
        ones[i, pl.ds(0, L)] = one

    plsc.subcore_barrier()

    @pl.loop(0, T)
    def _chunk(i):
      off = (i * NW + wk) * ch
      pltpu.sync_copy(src_hbm.at[pl.ds(off, ch)], idx_s)
      pltpu.sync_copy(dst_hbm.at[pl.ds(off, ch)], idx_d)
      pltpu.async_copy(x_hbm.at[idx_s], rows, sem).wait()
      pltpu.sync_copy(rows, acc.at[idx_d], add=True)
      if with_counts:
        pltpu.sync_copy(ones, cacc.at[idx_d], add=True)

    plsc.subcore_barrier()

    @pl.loop(0, R // WB)
    def _wb(k):
      pltpu.sync_copy(acc.at[pl.ds(base + k * WB, WB)], rows.at[pl.ds(0, WB)])
      pltpu.sync_copy(rows.at[pl.ds(0, WB)], psum.at[pl.ds(obase + k * WB, WB)])

    if with_counts:
      @pl.loop(0, R // WB)
      def _wbc(k):
        pltpu.sync_copy(cacc.at[pl.ds(base + k * WB, WB)],
                        ones.at[pl.ds(0, WB)])
        pltpu.sync_copy(ones.at[pl.ds(0, WB)],
                        pcnt.at[pl.ds(obase + k * WB, WB)])

  return pl.kernel(body, out_type=out_type, mesh=mesh,
                   scratch_types=scratch), N_ACC


def _make_counts(N, E_pad, ch):
  NW = NC * NS
  N_ACC = _ceil_to(N + 1, NS * L)
  R = N_ACC // NS
  T = E_pad // (NW * ch)
  WB = min(ch, 128)
  assert E_pad % (NW * ch) == 0 and R % WB == 0

  mesh = plsc.VectorSubcoreMesh(
      core_axis_name="c", subcore_axis_name="s",
      num_cores=NC, num_subcores=NS)

  out_type = [jax.ShapeDtypeStruct((NC * N_ACC, L), jnp.float32)]
  scratch = [
      pltpu.VMEM((ch,), jnp.int32),
      pltpu.VMEM((ch, L), jnp.float32),
      pltpu.VMEM_SHARED((N_ACC, L), jnp.float32),
  ]

  def body(dst_hbm, pcnt, idx_d, buf, cacc):
    c = lax.axis_index("c")
    s = lax.axis_index("s")
    wk = s * NC + c
    base = s * R
    obase = c * N_ACC + base
    zero = jnp.zeros((L,), jnp.float32)
    one = jnp.ones((L,), jnp.float32)

    @pl.loop(0, ch)
    def _zfill(i):
      buf[i, pl.ds(0, L)] = zero

    @pl.loop(0, R // WB)
    def _zero(k):
      pltpu.sync_copy(buf.at[pl.ds(0, WB)], cacc.at[pl.ds(base + k * WB, WB)])

    @pl.loop(0, ch)
    def _ofill(i):
      buf[i, pl.ds(0, L)] = one

    plsc.subcore_barrier()

    @pl.loop(0, T)
    def _chunk(i):
      off = (i * NW + wk) * ch
      pltpu.sync_copy(dst_hbm.at[pl.ds(off, ch)], idx_d)
      pltpu.sync_copy(buf, cacc.at[idx_d], add=True)

    plsc.subcore_barrier()

    @pl.loop(0, R // WB)
    def _wb(k):
      pltpu.sync_copy(cacc.at[pl.ds(base + k * WB, WB)], buf.at[pl.ds(0, WB)])
      pltpu.sync_copy(buf.at[pl.ds(0, WB)], pcnt.at[pl.ds(obase + k * WB, WB)])

  return pl.kernel(body, out_type=out_type, mesh=mesh,
                   scratch_types=scratch), N_ACC


def _dst_eff_body(ei_ref, out_ref, *, dummy):
  src = ei_ref[0]
  dst = ei_ref[1]
  out_ref[...] = jnp.where(src == dst, jnp.int32(dummy), dst)


def _layer1_body(x_ref, p_ref, c_ref, w_ref, b_ref, o_ref):
  s = p_ref[0] + p_ref[1]
  cnt = jnp.sum(c_ref[0] + c_ref[1], axis=1, keepdims=True) * (1.0 / L)
  mean = s * (1.0 / jnp.maximum(cnt, 1.0))
  d = x_ref.shape[1]
  h = (jnp.dot(x_ref[...], w_ref[:d], preferred_element_type=jnp.float32)
       + jnp.dot(mean, w_ref[d:], preferred_element_type=jnp.float32)
       + b_ref[...])
  h = jnp.maximum(h, 0.0)
  nrm = jnp.maximum(jnp.sqrt(jnp.sum(h * h, axis=1, keepdims=True)), 1e-12)
  o_ref[...] = h / nrm


def _layer2_body(h_ref, p_ref, c_ref, w_ref, b_ref,
                 wp1_ref, bp1_ref, wp2_ref, bp2_ref, o_ref):
  s = p_ref[0] + p_ref[1]
  cnt = jnp.sum(c_ref[0] + c_ref[1], axis=1, keepdims=True) * (1.0 / L)
  mean = s * (1.0 / jnp.maximum(cnt, 1.0))
  d = h_ref.shape[1]
  h = (jnp.dot(h_ref[...], w_ref[:d], preferred_element_type=jnp.float32)
       + jnp.dot(mean, w_ref[d:], preferred_element_type=jnp.float32)
       + b_ref[...])
  h = jnp.maximum(h, 0.0)
  nrm = jnp.maximum(jnp.sqrt(jnp.sum(h * h, axis=1, keepdims=True)), 1e-12)
  h = h / nrm
  z = jnp.dot(h, wp1_ref[...], preferred_element_type=jnp.float32) + bp1_ref[...]
  z = jnp.dot(z, wp2_ref[...], preferred_element_type=jnp.float32) + bp2_ref[...]
  m = jnp.max(z, axis=1, keepdims=True)
  z = z - m
  lse = jnp.log(jnp.sum(jnp.exp(z), axis=1, keepdims=True))
  o_ref[...] = z - lse


@jax.jit
def kernel(x, edge_index, batch, W1, b1, W2, b2, Wp1, bp1, Wp2, bp2):
  del batch
  N, D = x.shape
  E = edge_index.shape[1]
  E_pad = _ceil_to(E, NC * NS * CH)
  BN = 1000 if N % 1000 == 0 else N
  G = N // BN

  N_ACC = _ceil_to(N + 1, NS * L)
  P = E_pad - E
  pad = jnp.stack([
      jnp.zeros((P,), jnp.int32),
      N + (jnp.arange(P, dtype=jnp.int32) % (N_ACC - N)),
  ])
  ei = jnp.concatenate([edge_index, pad], axis=1)
  ei_blk = ei.reshape(2, E_pad // CH, CH)

  dst_eff = pl.pallas_call(
      functools.partial(_dst_eff_body, dummy=N),
      out_shape=jax.ShapeDtypeStruct((E_pad // CH, CH), jnp.int32),
  )(ei_blk)
  src = ei[0]
  dst_eff = dst_eff.reshape(E_pad)

  cntk, _ = _make_counts(N, E_pad, CH)
  (pcnt1,) = cntk(dst_eff)
  pcnt1 = pcnt1.reshape(NC, N_ACC, L)

  seg1, _ = _make_seg_sum(N, D, E_pad, CH, with_counts=False)
  (psum1,) = seg1(x, src, dst_eff)
  psum1 = psum1.reshape(NC, N_ACC, D)

  grid = (G,)
  h1 = pl.pallas_call(
      _layer1_body,
      grid=grid,
      in_specs=[
          pl.BlockSpec((BN, D), lambda i: (i, 0)),
          pl.BlockSpec((NC, BN, D), lambda i: (0, i, 0)),
          pl.BlockSpec((NC, BN, L), lambda i: (0, i, 0)),
          pl.BlockSpec((2 * D, D), lambda i: (0, 0)),
          pl.BlockSpec((1, D), lambda i: (0, 0)),
      ],
      out_specs=pl.BlockSpec((BN, D), lambda i: (i, 0)),
      out_shape=jax.ShapeDtypeStruct((N, D), jnp.float32),
  )(x, psum1, pcnt1, W1, b1.reshape(1, D))

  seg2, _ = _make_seg_sum(N, D, E_pad, CH, with_counts=False)
  (psum2,) = seg2(h1, src, dst_eff)
  psum2 = psum2.reshape(NC, N_ACC, D)

  out = pl.pallas_call(
      _layer2_body,
      grid=grid,
      in_specs=[
          pl.BlockSpec((BN, D), lambda i: (i, 0)),
          pl.BlockSpec((NC, BN, D), lambda i: (0, i, 0)),
          pl.BlockSpec((NC, BN, L), lambda i: (0, i, 0)),
          pl.BlockSpec((2 * D, D), lambda i: (0, 0)),
          pl.BlockSpec((1, D), lambda i: (0, 0)),
          pl.BlockSpec((D, D), lambda i: (0, 0)),
          pl.BlockSpec((1, D), lambda i: (0, 0)),
          pl.BlockSpec((D, D), lambda i: (0, 0)),
          pl.BlockSpec((1, D), lambda i: (0, 0)),
      ],
      out_specs=pl.BlockSpec((BN, D), lambda i: (i, 0)),
      out_shape=jax.ShapeDtypeStruct((N, D), jnp.float32),
  )(h1, psum2, pcnt1, W2, b2.reshape(1, D),
    Wp1, bp1.reshape(1, D), Wp2, bp2.reshape(1, D))

  return out


# speedup vs baseline: 1.0674x; 1.0674x over previous
import functools

import jax
import jax.numpy as jnp
from jax import lax
from jax.experimental import pallas as pl
from jax.experimental.pallas import tpu as pltpu
from jax.experimental.pallas import tpu_sc as plsc

NC = 2
NS = 16
L = 16
CH = 128


def _ceil_to(x, m):
  return (x + m - 1) // m * m


def _make_seg_sum(N, D, E_pad, ch, with_counts):
  NW = NC * NS
  N_ACC = _ceil_to(N + 1, NS * L)
  R = N_ACC // NS
  T = E_pad // (NW * ch)
  assert E_pad % (NW * ch) == 0

  mesh = plsc.VectorSubcoreMesh(
      core_axis_name="c", subcore_axis_name="s",
      num_cores=NC, num_subcores=NS)

  WB = min(ch, 128)
  assert R % WB == 0
  out_type = [jax.ShapeDtypeStruct((NC * N_ACC, D), jnp.float32)]
  scratch = [
      pltpu.VMEM((2, ch), jnp.int32),
      pltpu.VMEM((ch, D), jnp.float32),
      pltpu.VMEM_SHARED((N_ACC, D), jnp.float32),
      pltpu.SemaphoreType.DMA,
  ]
  if with_counts:
    out_type.append(jax.ShapeDtypeStruct((NC * N_ACC, L), jnp.float32))
    scratch += [
        pltpu.VMEM((ch, L), jnp.float32),
        pltpu.VMEM_SHARED((N_ACC, L), jnp.float32),
    ]

  def body(x_hbm, sd_hbm, *refs):
    if with_counts:
      (psum, pcnt, idx2, rows, acc, sem, ones, cacc) = refs
    else:
      (psum, idx2, rows, acc, sem) = refs
    c = lax.axis_index("c")
    s = lax.axis_index("s")
    wk = s * NC + c
    base = s * R
    obase = c * N_ACC + base
    zero = jnp.zeros((L,), jnp.float32)
    one = jnp.ones((L,), jnp.float32)

    @pl.loop(0, WB)
    def _zfill(i):
      for j in range(D // L):
        rows[i, pl.ds(j * L, L)] = zero

    @pl.loop(0, R // WB)
    def _zero(k):
      pltpu.sync_copy(rows.at[pl.ds(0, WB)], acc.at[pl.ds(base + k * WB, WB)])

    if with_counts:
      @pl.loop(0, ch)
      def _czfill(i):
        ones[i, pl.ds(0, L)] = zero

      @pl.loop(0, R // WB)
      def _czero(k):
        pltpu.sync_copy(ones.at[pl.ds(0, WB)],
                        cacc.at[pl.ds(base + k * WB, WB)])

      @pl.loop(0, ch)
      def _ofill(i):
        ones[i, pl.ds(0, L)] = one

    plsc.subcore_barrier()

    @pl.loop(0, T)
    def _chunk(i):
      blk = i * NW + wk
      pltpu.sync_copy(sd_hbm.at[blk], idx2)
      pltpu.async_copy(x_hbm.at[idx2.at[0]], rows, sem).wait()
      pltpu.sync_copy(rows, acc.at[idx2.at[1]], add=True)
      if with_counts:
        pltpu.sync_copy(ones, cacc.at[idx2.at[1]], add=True)

    plsc.subcore_barrier()

    @pl.loop(0, R // WB)
    def _wb(k):
      pltpu.sync_copy(acc.at[pl.ds(base + k * WB, WB)], rows.at[pl.ds(0, WB)])
      pltpu.sync_copy(rows.at[pl.ds(0, WB)], psum.at[pl.ds(obase + k * WB, WB)])

    if with_counts:
      @pl.loop(0, R // WB)
      def _wbc(k):
        pltpu.sync_copy(cacc.at[pl.ds(base + k * WB, WB)],
                        ones.at[pl.ds(0, WB)])
        pltpu.sync_copy(ones.at[pl.ds(0, WB)],
                        pcnt.at[pl.ds(obase + k * WB, WB)])

  return pl.kernel(body, out_type=out_type, mesh=mesh,
                   scratch_types=scratch), N_ACC


def _make_counts(N, E_pad, ch):
  NW = NC * NS
  N_ACC = _ceil_to(N + 1, NS * L)
  R = N_ACC // NS
  T = E_pad // (NW * ch)
  WB = min(ch, 128)
  assert E_pad % (NW * ch) == 0 and R % WB == 0

  mesh = plsc.VectorSubcoreMesh(
      core_axis_name="c", subcore_axis_name="s",
      num_cores=NC, num_subcores=NS)

  out_type = [jax.ShapeDtypeStruct((NC * N_ACC, L), jnp.float32)]
  scratch = [
      pltpu.VMEM((ch,), jnp.int32),
      pltpu.VMEM((ch, L), jnp.float32),
      pltpu.VMEM_SHARED((N_ACC, L), jnp.float32),
  ]

  def body(dst_hbm, pcnt, idx_d, buf, cacc):
    c = lax.axis_index("c")
    s = lax.axis_index("s")
    wk = s * NC + c
    base = s * R
    obase = c * N_ACC + base
    zero = jnp.zeros((L,), jnp.float32)
    one = jnp.ones((L,), jnp.float32)

    @pl.loop(0, ch)
    def _zfill(i):
      buf[i, pl.ds(0, L)] = zero

    @pl.loop(0, R // WB)
    def _zero(k):
      pltpu.sync_copy(buf.at[pl.ds(0, WB)], cacc.at[pl.ds(base + k * WB, WB)])

    @pl.loop(0, ch)
    def _ofill(i):
      buf[i, pl.ds(0, L)] = one

    plsc.subcore_barrier()

    @pl.loop(0, T)
    def _chunk(i):
      off = (i * NW + wk) * ch
      pltpu.sync_copy(dst_hbm.at[pl.ds(off, ch)], idx_d)
      pltpu.sync_copy(buf, cacc.at[idx_d], add=True)

    plsc.subcore_barrier()

    @pl.loop(0, R // WB)
    def _wb(k):
      pltpu.sync_copy(cacc.at[pl.ds(base + k * WB, WB)], buf.at[pl.ds(0, WB)])
      pltpu.sync_copy(buf.at[pl.ds(0, WB)], pcnt.at[pl.ds(obase + k * WB, WB)])

  return pl.kernel(body, out_type=out_type, mesh=mesh,
                   scratch_types=scratch), N_ACC


def _dst_eff_body(ei_ref, out_ref, *, dummy):
  src = ei_ref[0]
  dst = ei_ref[1]
  out_ref[...] = jnp.where(src == dst, jnp.int32(dummy), dst)


def _layer1_body(x_ref, p_ref, c_ref, w_ref, b_ref, o_ref):
  s = p_ref[0] + p_ref[1]
  cnt = jnp.sum(c_ref[0] + c_ref[1], axis=1, keepdims=True) * (1.0 / L)
  mean = s * (1.0 / jnp.maximum(cnt, 1.0))
  d = x_ref.shape[1]
  h = (jnp.dot(x_ref[...], w_ref[:d], preferred_element_type=jnp.float32)
       + jnp.dot(mean, w_ref[d:], preferred_element_type=jnp.float32)
       + b_ref[...])
  h = jnp.maximum(h, 0.0)
  nrm = jnp.maximum(jnp.sqrt(jnp.sum(h * h, axis=1, keepdims=True)), 1e-12)
  o_ref[...] = h / nrm


def _layer2_body(h_ref, p_ref, c_ref, w_ref, b_ref,
                 wp1_ref, bp1_ref, wp2_ref, bp2_ref, o_ref):
  s = p_ref[0] + p_ref[1]
  cnt = jnp.sum(c_ref[0] + c_ref[1], axis=1, keepdims=True) * (1.0 / L)
  mean = s * (1.0 / jnp.maximum(cnt, 1.0))
  d = h_ref.shape[1]
  h = (jnp.dot(h_ref[...], w_ref[:d], preferred_element_type=jnp.float32)
       + jnp.dot(mean, w_ref[d:], preferred_element_type=jnp.float32)
       + b_ref[...])
  h = jnp.maximum(h, 0.0)
  nrm = jnp.maximum(jnp.sqrt(jnp.sum(h * h, axis=1, keepdims=True)), 1e-12)
  h = h / nrm
  z = jnp.dot(h, wp1_ref[...], preferred_element_type=jnp.float32) + bp1_ref[...]
  z = jnp.dot(z, wp2_ref[...], preferred_element_type=jnp.float32) + bp2_ref[...]
  m = jnp.max(z, axis=1, keepdims=True)
  z = z - m
  lse = jnp.log(jnp.sum(jnp.exp(z), axis=1, keepdims=True))
  o_ref[...] = z - lse


@jax.jit
def kernel(x, edge_index, batch, W1, b1, W2, b2, Wp1, bp1, Wp2, bp2):
  del batch
  N, D = x.shape
  E = edge_index.shape[1]
  E_pad = _ceil_to(E, NC * NS * CH)
  BN = 1000 if N % 1000 == 0 else N
  G = N // BN

  N_ACC = _ceil_to(N + 1, NS * L)
  P = E_pad - E
  pad = jnp.stack([
      jnp.zeros((P,), jnp.int32),
      N + (jnp.arange(P, dtype=jnp.int32) % (N_ACC - N)),
  ])
  ei = jnp.concatenate([edge_index, pad], axis=1)
  ei_blk = ei.reshape(2, E_pad // CH, CH)
  src = ei_blk[0]

  dst_eff = pl.pallas_call(
      functools.partial(_dst_eff_body, dummy=N),
      out_shape=jax.ShapeDtypeStruct((E_pad // CH, CH), jnp.int32),
  )(ei_blk)
  sd = jnp.stack([src, dst_eff], axis=1)

  cntk, _ = _make_counts(N, E_pad, CH)
  (pcnt1,) = cntk(dst_eff.reshape(E_pad))
  pcnt1 = pcnt1.reshape(NC, N_ACC, L)

  seg1, _ = _make_seg_sum(N, D, E_pad, CH, with_counts=False)
  (psum1,) = seg1(x, sd)
  psum1 = psum1.reshape(NC, N_ACC, D)

  grid = (G,)
  h1 = pl.pallas_call(
      _layer1_body,
      grid=grid,
      in_specs=[
          pl.BlockSpec((BN, D), lambda i: (i, 0)),
          pl.BlockSpec((NC, BN, D), lambda i: (0, i, 0)),
          pl.BlockSpec((NC, BN, L), lambda i: (0, i, 0)),
          pl.BlockSpec((2 * D, D), lambda i: (0, 0)),
          pl.BlockSpec((1, D), lambda i: (0, 0)),
      ],
      out_specs=pl.BlockSpec((BN, D), lambda i: (i, 0)),
      out_shape=jax.ShapeDtypeStruct((N, D), jnp.float32),
  )(x, psum1, pcnt1, W1, b1.reshape(1, D))

  seg2, _ = _make_seg_sum(N, D, E_pad, CH, with_counts=False)
  (psum2,) = seg2(h1, sd)
  psum2 = psum2.reshape(NC, N_ACC, D)

  out = pl.pallas_call(
      _layer2_body,
      grid=grid,
      in_specs=[
          pl.BlockSpec((BN, D), lambda i: (i, 0)),
          pl.BlockSpec((NC, BN, D), lambda i: (0, i, 0)),
          pl.BlockSpec((NC, BN, L), lambda i: (0, i, 0)),
          pl.BlockSpec((2 * D, D), lambda i: (0, 0)),
          pl.BlockSpec((1, D), lambda i: (0, 0)),
          pl.BlockSpec((D, D), lambda i: (0, 0)),
          pl.BlockSpec((1, D), lambda i: (0, 0)),
          pl.BlockSpec((D, D), lambda i: (0, 0)),
          pl.BlockSpec((1, D), lambda i: (0, 0)),
      ],
      out_specs=pl.BlockSpec((BN, D), lambda i: (i, 0)),
      out_shape=jax.ShapeDtypeStruct((N, D), jnp.float32),
  )(h1, psum2, pcnt1, W2, b2.reshape(1, D),
    Wp1, bp1.reshape(1, D), Wp2, bp2.reshape(1, D))

  return out


# speedup vs baseline: 1.0677x; 1.0003x over previous
import functools

import jax
import jax.numpy as jnp
from jax import lax
from jax.experimental import pallas as pl
from jax.experimental.pallas import tpu as pltpu
from jax.experimental.pallas import tpu_sc as plsc

NC = 2
NS = 16
L = 16
CH = 128


def _ceil_to(x, m):
  return (x + m - 1) // m * m


def _make_seg_sum(N, D, E_pad, ch):
  NW = NC * NS
  N_ACC = _ceil_to(N + 1, NS * L)
  R = N_ACC // NS
  T = E_pad // (NW * ch)
  assert E_pad % (NW * ch) == 0

  mesh = plsc.VectorSubcoreMesh(
      core_axis_name="c", subcore_axis_name="s",
      num_cores=NC, num_subcores=NS)

  WB = min(ch, 128)
  assert R % WB == 0
  out_type = [jax.ShapeDtypeStruct((NC * N_ACC, D), jnp.float32)]
  scratch = [
      pltpu.VMEM((2, ch), jnp.int32),
      pltpu.VMEM((ch, D), jnp.float32),
      pltpu.VMEM_SHARED((N_ACC, D), jnp.float32),
      pltpu.SemaphoreType.DMA,
  ]

  def body(x_hbm, sd_hbm, psum, idx2, rows, acc, sem):
    c = lax.axis_index("c")
    s = lax.axis_index("s")
    wk = s * NC + c
    base = s * R
    obase = c * N_ACC + base
    zero = jnp.zeros((L,), jnp.float32)

    @pl.loop(0, WB)
    def _zfill(i):
      for j in range(D // L):
        rows[i, pl.ds(j * L, L)] = zero

    @pl.loop(0, R // WB)
    def _zero(k):
      pltpu.sync_copy(rows.at[pl.ds(0, WB)], acc.at[pl.ds(base + k * WB, WB)])

    plsc.subcore_barrier()

    @pl.loop(0, T)
    def _chunk(i):
      blk = i * NW + wk
      pltpu.sync_copy(sd_hbm.at[blk], idx2)
      pltpu.async_copy(x_hbm.at[idx2.at[0]], rows, sem).wait()
      pltpu.sync_copy(rows, acc.at[idx2.at[1]], add=True)

    plsc.subcore_barrier()

    @pl.loop(0, R // WB)
    def _wb(k):
      pltpu.sync_copy(acc.at[pl.ds(base + k * WB, WB)], rows.at[pl.ds(0, WB)])
      pltpu.sync_copy(rows.at[pl.ds(0, WB)], psum.at[pl.ds(obase + k * WB, WB)])

  return pl.kernel(body, out_type=out_type, mesh=mesh,
                   scratch_types=scratch), N_ACC


def _make_counts(N, E_pad, ch):
  NW = NC * NS
  N_ACC = _ceil_to(N + 1, NS * L)
  R = N_ACC // NS
  T = E_pad // (NW * ch)
  WB = min(ch, 128)
  assert E_pad % (NW * ch) == 0 and R % WB == 0

  mesh = plsc.VectorSubcoreMesh(
      core_axis_name="c", subcore_axis_name="s",
      num_cores=NC, num_subcores=NS)

  out_type = [jax.ShapeDtypeStruct((NC * N_ACC, L), jnp.float32)]
  scratch = [
      pltpu.VMEM((ch,), jnp.int32),
      pltpu.VMEM((ch, L), jnp.float32),
      pltpu.VMEM_SHARED((N_ACC, L), jnp.float32),
  ]

  def body(dst_hbm, pcnt, idx_d, buf, cacc):
    c = lax.axis_index("c")
    s = lax.axis_index("s")
    wk = s * NC + c
    base = s * R
    obase = c * N_ACC + base
    zero = jnp.zeros((L,), jnp.float32)
    one = jnp.ones((L,), jnp.float32)

    @pl.loop(0, ch)
    def _zfill(i):
      buf[i, pl.ds(0, L)] = zero

    @pl.loop(0, R // WB)
    def _zero(k):
      pltpu.sync_copy(buf.at[pl.ds(0, WB)], cacc.at[pl.ds(base + k * WB, WB)])

    @pl.loop(0, ch)
    def _ofill(i):
      buf[i, pl.ds(0, L)] = one

    plsc.subcore_barrier()

    @pl.loop(0, T)
    def _chunk(i):
      off = (i * NW + wk) * ch
      pltpu.sync_copy(dst_hbm.at[pl.ds(off, ch)], idx_d)
      pltpu.sync_copy(buf, cacc.at[idx_d], add=True)

    plsc.subcore_barrier()

    @pl.loop(0, R // WB)
    def _wb(k):
      pltpu.sync_copy(cacc.at[pl.ds(base + k * WB, WB)], buf.at[pl.ds(0, WB)])
      pltpu.sync_copy(buf.at[pl.ds(0, WB)], pcnt.at[pl.ds(obase + k * WB, WB)])

  return pl.kernel(body, out_type=out_type, mesh=mesh,
                   scratch_types=scratch), N_ACC


def _dst_eff_body(ei_ref, out_ref, *, dummy):
  src = ei_ref[0]
  dst = ei_ref[1]
  out_ref[...] = jnp.where(src == dst, jnp.int32(dummy), dst)


def _layer1_body(x_ref, p_ref, c_ref, w_ref, b_ref, o_ref):
  s = p_ref[0] + p_ref[1]
  cnt = jnp.sum(c_ref[0] + c_ref[1], axis=1, keepdims=True) * (1.0 / L)
  mean = s * (1.0 / jnp.maximum(cnt, 1.0))
  d = x_ref.shape[1]
  h = (jnp.dot(x_ref[...], w_ref[:d], preferred_element_type=jnp.float32)
       + jnp.dot(mean, w_ref[d:], preferred_element_type=jnp.float32)
       + b_ref[...])
  h = jnp.maximum(h, 0.0)
  nrm = jnp.maximum(jnp.sqrt(jnp.sum(h * h, axis=1, keepdims=True)), 1e-12)
  o_ref[...] = h / nrm


def _layer2_body(h_ref, p_ref, c_ref, w_ref, b_ref,
                 wp1_ref, bp1_ref, wp2_ref, bp2_ref, o_ref):
  s = p_ref[0] + p_ref[1]
  cnt = jnp.sum(c_ref[0] + c_ref[1], axis=1, keepdims=True) * (1.0 / L)
  mean = s * (1.0 / jnp.maximum(cnt, 1.0))
  d = h_ref.shape[1]
  h = (jnp.dot(h_ref[...], w_ref[:d], preferred_element_type=jnp.float32)
       + jnp.dot(mean, w_ref[d:], preferred_element_type=jnp.float32)
       + b_ref[...])
  h = jnp.maximum(h, 0.0)
  nrm = jnp.maximum(jnp.sqrt(jnp.sum(h * h, axis=1, keepdims=True)), 1e-12)
  h = h / nrm
  z = jnp.dot(h, wp1_ref[...], preferred_element_type=jnp.float32) + bp1_ref[...]
  z = jnp.dot(z, wp2_ref[...], preferred_element_type=jnp.float32) + bp2_ref[...]
  m = jnp.max(z, axis=1, keepdims=True)
  z = z - m
  lse = jnp.log(jnp.sum(jnp.exp(z), axis=1, keepdims=True))
  o_ref[...] = z - lse


@jax.jit
def kernel(x, edge_index, batch, W1, b1, W2, b2, Wp1, bp1, Wp2, bp2):
  del batch
  N, D = x.shape
  E = edge_index.shape[1]
  E_pad = _ceil_to(E, NC * NS * CH)
  BN = 1000 if N % 1000 == 0 else N
  G = N // BN

  N_ACC = _ceil_to(N + 1, NS * L)
  P = E_pad - E
  pad = jnp.stack([
      jnp.zeros((P,), jnp.int32),
      N + (jnp.arange(P, dtype=jnp.int32) % (N_ACC - N)),
  ])
  ei = jnp.concatenate([edge_index, pad], axis=1)
  ei_blk = ei.reshape(2, E_pad // CH, CH)
  src = ei_blk[0]

  dst_eff = pl.pallas_call(
      functools.partial(_dst_eff_body, dummy=N),
      out_shape=jax.ShapeDtypeStruct((E_pad // CH, CH), jnp.int32),
  )(ei_blk)
  sd = jnp.stack([src, dst_eff], axis=1)

  cntk, _ = _make_counts(N, E_pad, CH)
  (pcnt1,) = cntk(dst_eff.reshape(E_pad))
  pcnt1 = pcnt1.reshape(NC, N_ACC, L)

  seg1, _ = _make_seg_sum(N, D, E_pad, CH)
  (psum1,) = seg1(x, sd)
  psum1 = psum1.reshape(NC, N_ACC, D)

  grid = (G,)
  h1 = pl.pallas_call(
      _layer1_body,
      grid=grid,
      in_specs=[
          pl.BlockSpec((BN, D), lambda i: (i, 0)),
          pl.BlockSpec((NC, BN, D), lambda i: (0, i, 0)),
          pl.BlockSpec((NC, BN, L), lambda i: (0, i, 0)),
          pl.BlockSpec((2 * D, D), lambda i: (0, 0)),
          pl.BlockSpec((1, D), lambda i: (0, 0)),
      ],
      out_specs=pl.BlockSpec((BN, D), lambda i: (i, 0)),
      out_shape=jax.ShapeDtypeStruct((N, D), jnp.float32),
  )(x, psum1, pcnt1, W1, b1.reshape(1, D))

  seg2, _ = _make_seg_sum(N, D, E_pad, CH)
  (psum2,) = seg2(h1, sd)
  psum2 = psum2.reshape(NC, N_ACC, D)

  out = pl.pallas_call(
      _layer2_body,
      grid=grid,
      in_specs=[
          pl.BlockSpec((BN, D), lambda i: (i, 0)),
          pl.BlockSpec((NC, BN, D), lambda i: (0, i, 0)),
          pl.BlockSpec((NC, BN, L), lambda i: (0, i, 0)),
          pl.BlockSpec((2 * D, D), lambda i: (0, 0)),
          pl.BlockSpec((1, D), lambda i: (0, 0)),
          pl.BlockSpec((D, D), lambda i: (0, 0)),
          pl.BlockSpec((1, D), lambda i: (0, 0)),
          pl.BlockSpec((D, D), lambda i: (0, 0)),
          pl.BlockSpec((1, D), lambda i: (0, 0)),
      ],
      out_specs=pl.BlockSpec((BN, D), lambda i: (i, 0)),
      out_shape=jax.ShapeDtypeStruct((N, D), jnp.float32),
  )(h1, psum2, pcnt1, W2, b2.reshape(1, D),
    Wp1, bp1.reshape(1, D), Wp2, bp2.reshape(1, D))

  return out
